# Initial kernel scaffold; baseline (speedup 1.0000x reference)
import jax
import jax.numpy as jnp
from jax.experimental import pallas as pl


def kernel(solvent_x, solvent_edge_index, solvent_edge_attr, solvent_batch, solvent_ap, solvent_bp, solvent_topopsa, solvent_inter_hb, solute_x, solute_edge_index, solute_edge_attr, solute_batch, solute_ap, solute_bp, solute_topopsa, solute_inter_hb, T, params):
    raise NotImplementedError("write your pallas kernel here")


# speedup vs baseline: 4.8831x; 4.8831x over previous
import functools

import jax
import jax.numpy as jnp
from jax import lax
from jax.experimental import pallas as pl
from jax.experimental.pallas import tpu as pltpu
from jax.experimental.pallas import tpu_sc as plsc

V_IN = 128
E_IN = 16
H = 64
NG = 256
N1 = 10000
N2 = 20000
N2P = 20480
E1 = 320000
E2 = 640000
K = 128
CH = 157
EPT = K * CH
NTILES = 32
E2P = EPT * NTILES
NB = N2P // 1024
EB = 4096
NBE = E2P // EB
STRIPE = N2P // 16


def _bspec(shape, imap):
    return pl.BlockSpec(shape, imap)


def _full(shape):
    return pl.BlockSpec(shape, lambda i: tuple(0 for _ in shape))


def _pre_body(x_ref, b_ref, u8_ref, w1a_ref, w1b_ref, w1u_ref, b1_ref,
              *ab_ref):
    bb = b_ref[...]
    oneh = (bb == lax.broadcasted_iota(jnp.int32, (1024, NG), 1).astype(jnp.float32)).astype(jnp.float32)
    u1 = _dot(u8_ref[...], w1u_ref[...]) + b1_ref[...]
    x = x_ref[...]
    ab_ref[0][...] = _dot(x, w1a_ref[...]) + \
        _dot(oneh, u1)
    ab_ref[1][...] = _dot(x, w1b_ref[...])


def _tc_pre(x, batchf2d, u8, w1a, w1b, w1u8, b1r):
    return pl.pallas_call(
        _pre_body,
        grid=(NB,),
        in_specs=[
            _bspec((1024, V_IN), lambda i: (i, 0)),
            _bspec((1024, 1), lambda i: (i, 0)),
            _full((NG, 8)), _full((V_IN, H)), _full((V_IN, H)),
            _full((8, H)), _full((1, H)),
        ],
        out_shape=[jax.ShapeDtypeStruct((N2P, H), jnp.float32),
                   jax.ShapeDtypeStruct((N2P, H), jnp.float32)],
        out_specs=[_bspec((1024, H), lambda i: (i, 0)),
                   _bspec((1024, H), lambda i: (i, 0))],
    )(x, batchf2d, u8, w1a, w1b, w1u8, b1r)


def _c1_body(ea_ref, w_ref, o_ref):
    o_ref[...] = _dot(ea_ref[...], w_ref[...])


def _tc_c1(ea, w1c):
    return pl.pallas_call(
        _c1_body,
        grid=(NBE,),
        in_specs=[_bspec((EB, E_IN), lambda i: (i, 0)), _full((E_IN, H))],
        out_shape=jax.ShapeDtypeStruct((E2P, H), jnp.float32),
        out_specs=_bspec((EB, H), lambda i: (i, 0)),
    )(ea, w1c)


def _r1m_body(r_ref, w21_ref, w2c_ref, o_ref):
    m2 = _dot(w21_ref[...], w2c_ref[...])
    o_ref[...] = _dot(r_ref[...], m2)


def _tc_r1m(r1, w21, w2c):
    return pl.pallas_call(
        _r1m_body,
        grid=(NBE,),
        in_specs=[_bspec((EB, H), lambda i: (i, 0)), _full((H, H)), _full((H, H))],
        out_shape=jax.ShapeDtypeStruct((E2P, H), jnp.float32),
        out_specs=_bspec((EB, H), lambda i: (i, 0)),
    )(r1, w21, w2c)


def _dot(a, b):
    return jnp.dot(a, b, preferred_element_type=jnp.float32,
                   precision=lax.Precision.HIGHEST)


def _dotT(a, b):
    return lax.dot_general(a, b, (((0,), (0,)), ((), ())),
                           preferred_element_type=jnp.float32,
                           precision=lax.Precision.HIGHEST)


def _stage1_body(x_ref, b_ref, s_ref, cnt_ref, u8_ref,
                 w21_ref, b21_ref,
                 wnx_ref, wna_ref, wnu_ref, bn_ref, wn2_ref, bn2_ref,
                 wgu_ref, wgn_ref, wge_ref, bg_ref, wg2_ref, bg2_ref,
                 ms_ref, gw_ref,
                 w2u_ref, b2e_ref, w2c_ref,
                 x1_ref, msc_ref, rw_ref, u1_ref, u2t_ref, gst_ref,
                 acc_sum, acc_sq, acc_eg, acc_g):
    i = pl.program_id(0)

    @pl.when(i == 0)
    def _():
        acc_sum[...] = jnp.zeros_like(acc_sum)
        acc_sq[...] = jnp.zeros_like(acc_sq)
        acc_eg[...] = jnp.zeros_like(acc_eg)
        acc_g[...] = jnp.zeros_like(acc_g)

    bb = b_ref[...]
    oneh = (bb == lax.broadcasted_iota(jnp.int32, (1024, NG), 1).astype(jnp.float32)).astype(jnp.float32)
    s = s_ref[0] + s_ref[1]
    cnt = (cnt_ref[0, :, 0:1] + cnt_ref[1, :, 0:1])
    agg = _dot(s, w21_ref[...]) + cnt * b21_ref[...]
    un = _dot(u8_ref[...], wnu_ref[...]) + bn_ref[...]
    nh = jnp.maximum(_dot(x_ref[...], wnx_ref[...]) + _dot(agg, wna_ref[...]) +
                     _dot(oneh, un), 0.0)
    x1 = _dot(nh, wn2_ref[...]) + bn2_ref[...]
    x1_ref[...] = x1

    acc_sum[...] += _dotT(oneh, x1)
    acc_sq[...] += _dotT(oneh, x1 * x1)
    acc_eg[...] += _dotT(oneh, s)
    ones_col = jnp.ones((1024, 1), jnp.float32)
    acc_g[:, 0:1] += _dotT(oneh, cnt)
    acc_g[:, 1:2] += _dotT(oneh, ones_col)

    @pl.when(i == NB - 1)
    def _():
        ecnt = acc_g[:, 0:1]
        ncnt = jnp.maximum(acc_g[:, 1:2], 1.0)
        m = acc_sum[...] / ncnt
        q = acc_sq[...] / ncnt
        ms = ms_ref[...]
        var = q - (2.0 * ms - ms * ms) * m * m
        rstd = lax.rsqrt(var + 1e-5)
        edge_agg = (_dot(acc_eg[...], w21_ref[...]) + ecnt * b21_ref[...]) / \
            jnp.maximum(ecnt, 1.0)
        gh = jnp.maximum(_dot(u8_ref[...], wgu_ref[...]) + _dot(m, wgn_ref[...]) +
                         _dot(edge_agg, wge_ref[...]) + bg_ref[...], 0.0)
        u1 = _dot(gh, wg2_ref[...]) + bg2_ref[...]
        u1_ref[...] = u1
        msc_ref[...] = m * ms
        rw_ref[...] = rstd * gw_ref[...]
        u2t_ref[...] = _dot(u1, w2u_ref[...]) + b2e_ref[...] + \
            _dot(b21_ref[...], w2c_ref[...])
        gst_ref[...] = acc_g[...]


def _tc_stage1(x, batchf2d, s2p, cnt2p, u8, p8):
    outs = pl.pallas_call(
        _stage1_body,
        grid=(NB,),
        in_specs=[
            _bspec((1024, V_IN), lambda i: (i, 0)),
            _bspec((1024, 1), lambda i: (i, 0)),
            _bspec((2, 1024, H), lambda i: (0, i, 0)),
            _bspec((2, 1024, 16), lambda i: (0, i, 0)),
            _full((NG, 8)),
            _full((H, H)), _full((1, H)),
            _full((V_IN, H)), _full((H, H)), _full((8, H)), _full((1, H)),
            _full((H, H)), _full((1, H)),
            _full((8, H)), _full((H, H)), _full((H, H)), _full((1, H)),
            _full((H, H)), _full((1, H)),
            _full((1, H)), _full((1, H)),
            _full((H, H)), _full((1, H)), _full((H, H)),
        ],
        out_shape=[jax.ShapeDtypeStruct((N2P, H), jnp.float32),
                   jax.ShapeDtypeStruct((NG, H), jnp.float32),
                   jax.ShapeDtypeStruct((NG, H), jnp.float32),
                   jax.ShapeDtypeStruct((NG, H), jnp.float32),
                   jax.ShapeDtypeStruct((NG, H), jnp.float32),
                   jax.ShapeDtypeStruct((NG, 128), jnp.float32)],
        out_specs=[_bspec((1024, H), lambda i: (i, 0)),
                   _full((NG, H)), _full((NG, H)), _full((NG, H)),
                   _full((NG, H)), _full((NG, 128))],
        scratch_shapes=[pltpu.VMEM((NG, H), jnp.float32),
                        pltpu.VMEM((NG, H), jnp.float32),
                        pltpu.VMEM((NG, H), jnp.float32),
                        pltpu.VMEM((NG, 128), jnp.float32)],
    )(x, batchf2d, s2p, cnt2p, u8, *p8)
    return outs


def _stage1b_body(x1_ref, b_ref, msc_ref, rw_ref, bgn_ref, w2a_ref, w2b_ref,
                  u2t_ref, xn1_ref, *ab2_ref):
    bb = b_ref[...]
    oneh = (bb == lax.broadcasted_iota(jnp.int32, (1024, NG), 1).astype(jnp.float32)).astype(jnp.float32)
    xn1 = (x1_ref[...] - _dot(oneh, msc_ref[...])) * _dot(oneh, rw_ref[...]) + \
        bgn_ref[...]
    xn1_ref[...] = xn1
    ab2_ref[0][...] = _dot(xn1, w2a_ref[...]) + _dot(oneh, u2t_ref[...])
    ab2_ref[1][...] = _dot(xn1, w2b_ref[...])


def _tc_stage1b(x1, batchf2d, msc, rw, bgn, w2a, w2b, u2t):
    return pl.pallas_call(
        _stage1b_body,
        grid=(NB,),
        in_specs=[
            _bspec((1024, H), lambda i: (i, 0)),
            _bspec((1024, 1), lambda i: (i, 0)),
            _full((NG, H)), _full((NG, H)), _full((1, H)),
            _full((H, H)), _full((H, H)), _full((NG, H)),
        ],
        out_shape=[jax.ShapeDtypeStruct((N2P, H), jnp.float32),
                   jax.ShapeDtypeStruct((N2P, H), jnp.float32),
                   jax.ShapeDtypeStruct((N2P, H), jnp.float32)],
        out_specs=[_bspec((1024, H), lambda i: (i, 0)),
                   _bspec((1024, H), lambda i: (i, 0)),
                   _bspec((1024, H), lambda i: (i, 0))],
    )(x1, batchf2d, msc, rw, bgn, w2a, w2b, u2t)


def _stage2_body(xn1_ref, b_ref, s_ref, cnt_ref, u1_ref, gst_ref,
                 w22_ref, b22_ref,
                 wnx_ref, wna_ref, wnu_ref, bn_ref, wn2_ref, bn2_ref,
                 wgu_ref, wgn_ref, wge_ref, bg_ref, wg2_ref, bg2_ref,
                 ms_ref, gw_ref,
                 x2_ref, msc_ref, rw_ref, u2_ref,
                 acc_sum, acc_sq, acc_eg):
    i = pl.program_id(0)

    @pl.when(i == 0)
    def _():
        acc_sum[...] = jnp.zeros_like(acc_sum)
        acc_sq[...] = jnp.zeros_like(acc_sq)
        acc_eg[...] = jnp.zeros_like(acc_eg)

    bb = b_ref[...]
    oneh = (bb == lax.broadcasted_iota(jnp.int32, (1024, NG), 1).astype(jnp.float32)).astype(jnp.float32)
    s = s_ref[0] + s_ref[1]
    cnt = (cnt_ref[0, :, 0:1] + cnt_ref[1, :, 0:1])
    agg = _dot(s, w22_ref[...]) + cnt * b22_ref[...]
    un = _dot(u1_ref[...], wnu_ref[...]) + bn_ref[...]
    nh = jnp.maximum(_dot(xn1_ref[...], wnx_ref[...]) + _dot(agg, wna_ref[...]) +
                     _dot(oneh, un), 0.0)
    x2 = _dot(nh, wn2_ref[...]) + bn2_ref[...]
    x2_ref[...] = x2

    acc_sum[...] += _dotT(oneh, x2)
    acc_sq[...] += _dotT(oneh, x2 * x2)
    acc_eg[...] += _dotT(oneh, s)

    @pl.when(i == NB - 1)
    def _():
        ecnt = gst_ref[:, 0:1]
        ncnt = jnp.maximum(gst_ref[:, 1:2], 1.0)
        m = acc_sum[...] / ncnt
        q = acc_sq[...] / ncnt
        ms = ms_ref[...]
        var = q - (2.0 * ms - ms * ms) * m * m
        rstd = lax.rsqrt(var + 1e-5)
        edge_agg = (_dot(acc_eg[...], w22_ref[...]) + ecnt * b22_ref[...]) / \
            jnp.maximum(ecnt, 1.0)
        gh = jnp.maximum(_dot(u1_ref[...], wgu_ref[...]) + _dot(m, wgn_ref[...]) +
                         _dot(edge_agg, wge_ref[...]) + bg_ref[...], 0.0)
        u2 = _dot(gh, wg2_ref[...]) + bg2_ref[...]
        u2_ref[...] = u2
        msc_ref[...] = m * ms
        rw_ref[...] = rstd * gw_ref[...]


def _tc_stage2(xn1, batchf2d, s2p, cnt2p, u1, gstats, p8):
    return pl.pallas_call(
        _stage2_body,
        grid=(NB,),
        in_specs=[
            _bspec((1024, H), lambda i: (i, 0)),
            _bspec((1024, 1), lambda i: (i, 0)),
            _bspec((2, 1024, H), lambda i: (0, i, 0)),
            _bspec((2, 1024, 16), lambda i: (0, i, 0)),
            _full((NG, H)), _full((NG, 128)),
            _full((H, H)), _full((1, H)),
            _full((H, H)), _full((H, H)), _full((H, H)), _full((1, H)),
            _full((H, H)), _full((1, H)),
            _full((H, H)), _full((H, H)), _full((H, H)), _full((1, H)),
            _full((H, H)), _full((1, H)),
            _full((1, H)), _full((1, H)),
        ],
        out_shape=[jax.ShapeDtypeStruct((N2P, H), jnp.float32),
                   jax.ShapeDtypeStruct((NG, H), jnp.float32),
                   jax.ShapeDtypeStruct((NG, H), jnp.float32),
                   jax.ShapeDtypeStruct((NG, H), jnp.float32)],
        out_specs=[_bspec((1024, H), lambda i: (i, 0)),
                   _full((NG, H)), _full((NG, H)), _full((NG, H))],
        scratch_shapes=[pltpu.VMEM((NG, H), jnp.float32),
                        pltpu.VMEM((NG, H), jnp.float32),
                        pltpu.VMEM((NG, H), jnp.float32)],
    )(xn1, batchf2d, s2p, cnt2p, u1, gstats, *p8)


def _stage2b_body(x2_ref, b_ref, msc_ref, rw_ref, bgn_ref, u2_ref, gst_ref,
                  nf_ref, acc_xg):
    i = pl.program_id(0)

    @pl.when(i == 0)
    def _():
        acc_xg[...] = jnp.zeros_like(acc_xg)

    bb = b_ref[...]
    oneh = (bb == lax.broadcasted_iota(jnp.int32, (1024, NG), 1).astype(jnp.float32)).astype(jnp.float32)
    xn2 = (x2_ref[...] - _dot(oneh, msc_ref[...])) * _dot(oneh, rw_ref[...]) + \
        bgn_ref[...]
    acc_xg[...] += _dotT(oneh, xn2)

    @pl.when(i == NB - 1)
    def _():
        ncnt = jnp.maximum(gst_ref[:, 1:2], 1.0)
        xg = acc_xg[...] / ncnt
        nf_ref[:, 0:H] = xg
        nf_ref[:, H:2 * H] = u2_ref[...]


def _tc_stage2b(x2, batchf2d, msc2, rw2, bgn2, u2, gstats):
    return pl.pallas_call(
        _stage2b_body,
        grid=(NB,),
        in_specs=[
            _bspec((1024, H), lambda i: (i, 0)),
            _bspec((1024, 1), lambda i: (i, 0)),
            _full((NG, H)), _full((NG, H)), _full((1, H)),
            _full((NG, H)), _full((NG, 128)),
        ],
        out_shape=jax.ShapeDtypeStruct((NG, 128), jnp.float32),
        out_specs=_full((NG, 128)),
        scratch_shapes=[pltpu.VMEM((NG, H), jnp.float32)],
    )(x2, batchf2d, msc2, rw2, bgn2, u2, gstats)


def _system_body(nf_ref, sih_ref, tih_ref,
                 pw_ref, pb_ref, e1w_ref, e1b_ref, e2w_ref, e2br_ref,
                 root_ref, mb_ref, wih_ref, whh_ref, bih_ref, bhh_ref,
                 out_ref):
    D = 128
    h = jnp.maximum(_dot(nf_ref[...], pw_ref[...]) + pb_ref[...], 0.0)
    efa = jnp.concatenate([sih_ref[...], sih_ref[...]], axis=0)
    efb = jnp.concatenate([sih_ref[...], tih_ref[...]], axis=0)
    eha = jnp.maximum(efa * e1w_ref[...] + e1b_ref[...], 0.0)
    ehb = jnp.maximum(efb * e1w_ref[...] + e1b_ref[...], 0.0)
    hb = _dot(h, e2br_ref[...])
    msga = hb
    msgb = hb
    for k in range(32):
        qk = _dot(h, e2w_ref[pl.ds(k * D, D), :])
        msga = msga + eha[:, k:k + 1] * qk
        msgb = msgb + ehb[:, k:k + 1] * qk
    aggm = jnp.concatenate([msga[128:256], msga[0:128]], axis=0) + msgb
    out = jnp.maximum(_dot(h, root_ref[...]) + aggm + mb_ref[...], 0.0)
    gi = lax.dot_general(out, wih_ref[...], (((1,), (1,)), ((), ())),
                         preferred_element_type=jnp.float32,
                         precision=lax.Precision.HIGHEST) + bih_ref[...]
    gh = lax.dot_general(h, whh_ref[...], (((1,), (1,)), ((), ())),
                         preferred_element_type=jnp.float32,
                         precision=lax.Precision.HIGHEST) + bhh_ref[...]
    r = jax.nn.sigmoid(gi[:, 0:D] + gh[:, 0:D])
    z = jax.nn.sigmoid(gi[:, D:2 * D] + gh[:, D:2 * D])
    nh = jnp.tanh(gi[:, 2 * D:3 * D] + r * gh[:, 2 * D:3 * D])
    out_ref[...] = (1.0 - z) * nh + z * h


def _tc_system(nf, sih2, tih2, pw, pb, e1w, e1b, e2w, e2br, root, mb,
               wih, whh, bih, bhh):
    return pl.pallas_call(
        _system_body,
        grid=(1,),
        in_specs=[
            _full((NG, 128)), _full((128, 1)), _full((128, 1)),
            _full((128, 128)), _full((1, 128)), _full((1, 32)), _full((1, 32)),
            _full((4096, 128)), _full((128, 128)),
            _full((128, 128)), _full((1, 128)),
            _full((384, 128)), _full((384, 128)), _full((1, 384)), _full((1, 384)),
        ],
        out_shape=jax.ShapeDtypeStruct((NG, 128), jnp.float32),
        out_specs=_full((NG, 128)),
    )(nf, sih2, tih2, pw, pb, e1w, e1b, e2w, e2br, root, mb, wih, whh, bih, bhh)


@functools.lru_cache(maxsize=None)
def _make_sc_fused(store_r: bool):
    mesh = plsc.VectorSubcoreMesh(core_axis_name="c", subcore_axis_name="s")
    outs = [jax.ShapeDtypeStruct((2, N2P, H), jnp.float32)]
    if store_r:
        outs.append(jax.ShapeDtypeStruct((E2P, H), jnp.float32))

    @functools.partial(
        pl.kernel, mesh=mesh, out_type=outs,
        compiler_params=pltpu.CompilerParams(use_tc_tiling_on_sc=False),
        scratch_types=[
            pltpu.VMEM((K,), jnp.int32),
            pltpu.VMEM((K,), jnp.int32),
            pltpu.VMEM((K, H), jnp.float32),
            pltpu.VMEM((K, H), jnp.float32),
            pltpu.VMEM((K, H), jnp.float32),
            pltpu.VMEM_SHARED((N2P, H), jnp.float32),
            pltpu.SemaphoreType.DMA,
        ])
    def sc_fused(a_hbm, b_hbm, c_hbm, row_hbm, col_hbm, *rest):
        it = iter(rest)
        s_out = next(it)
        r_out = next(it) if store_r else None
        idx_r = next(it); idx_c = next(it)
        abuf = next(it); bbuf = next(it); obuf = next(it)
        s_sh = next(it); sem = next(it)

        c = lax.axis_index("c")
        sid = lax.axis_index("s")
        wid = sid * 2 + c

        def zrow(i, _):
            for j in range(H // 16):
                obuf[i, pl.ds(j * 16, 16)] = jnp.zeros((16,), jnp.float32)
            return 0
        lax.fori_loop(0, K, zrow, 0)
        r0 = sid * STRIPE
        for t in range(STRIPE // K):
            pltpu.sync_copy(obuf, s_sh.at[pl.ds(r0 + t * K, K)])
        plsc.subcore_barrier()

        def chunk(ch, _):
            base = pl.multiple_of(wid * EPT + ch * K, K)
            pltpu.sync_copy(row_hbm.at[pl.ds(base, K)], idx_r)
            pltpu.sync_copy(col_hbm.at[pl.ds(base, K)], idx_c)
            pltpu.async_copy(a_hbm.at[idx_r], abuf, sem).wait()
            pltpu.async_copy(b_hbm.at[idx_c], bbuf, sem).wait()
            pltpu.sync_copy(c_hbm.at[pl.ds(base, K)], obuf)

            def erow(i, _):
                for j in range(H // 16):
                    sl = pl.ds(j * 16, 16)
                    obuf[i, sl] = jnp.maximum(
                        abuf[i, sl] + bbuf[i, sl] + obuf[i, sl], 0.0)
                return 0
            lax.fori_loop(0, K, erow, 0)
            if store_r:
                pltpu.sync_copy(obuf, r_out.at[pl.ds(base, K)])
            pltpu.sync_copy(obuf, s_sh.at[idx_c], add=True)
            return 0
        lax.fori_loop(0, CH, chunk, 0)

        plsc.subcore_barrier()
        pltpu.sync_copy(s_sh.at[pl.ds(r0, STRIPE)], s_out.at[c, pl.ds(r0, STRIPE)])

    return sc_fused


@functools.lru_cache(maxsize=None)
def _make_sc_cnt():
    mesh = plsc.VectorSubcoreMesh(core_axis_name="c", subcore_axis_name="s")

    @functools.partial(
        pl.kernel, mesh=mesh,
        out_type=jax.ShapeDtypeStruct((2, N2P, 16), jnp.float32),
        compiler_params=pltpu.CompilerParams(use_tc_tiling_on_sc=False),
        scratch_types=[
            pltpu.VMEM((K,), jnp.int32),
            pltpu.VMEM((K, 16), jnp.float32),
            pltpu.VMEM((K, 16), jnp.float32),
            pltpu.VMEM_SHARED((N2P, 16), jnp.float32),
        ])
    def sc_cnt(col_hbm, cnt_out, idx_c, ones, zc, cnt_sh):
        c = lax.axis_index("c")
        sid = lax.axis_index("s")
        wid = sid * 2 + c

        def zrow(i, _):
            zc[i, pl.ds(0, 16)] = jnp.zeros((16,), jnp.float32)
            ones[i, pl.ds(0, 16)] = jnp.full((16,), 1.0, jnp.float32)
            return 0
        lax.fori_loop(0, K, zrow, 0)
        r0 = sid * STRIPE
        for t in range(STRIPE // K):
            pltpu.sync_copy(zc, cnt_sh.at[pl.ds(r0 + t * K, K)])
        plsc.subcore_barrier()

        def chunk(ch, _):
            base = pl.multiple_of(wid * EPT + ch * K, K)
            pltpu.sync_copy(col_hbm.at[pl.ds(base, K)], idx_c)
            pltpu.sync_copy(ones, cnt_sh.at[idx_c], add=True)
            return 0
        lax.fori_loop(0, CH, chunk, 0)

        plsc.subcore_barrier()
        pltpu.sync_copy(cnt_sh.at[pl.ds(r0, STRIPE)],
                        cnt_out.at[c, pl.ds(r0, STRIPE)])

    return sc_cnt


def _sc_pass1(a, b, cc, row, col):
    s, r = _make_sc_fused(True)(a, b, cc, row, col)
    return s, r


def _sc_pass2(a, b, cc, row, col):
    s, _ = _make_sc_fused(True)(a, b, cc, row, col)
    return s


def _sc_cnt(col):
    return _make_sc_cnt()(col)


def _pad8(w):
    return jnp.pad(w, ((0, 8 - w.shape[0]), (0, 0)))


def kernel(solvent_x, solvent_edge_index, solvent_edge_attr, solvent_batch,
           solvent_ap, solvent_bp, solvent_topopsa, solvent_inter_hb,
           solute_x, solute_edge_index, solute_edge_attr, solute_batch,
           solute_ap, solute_bp, solute_topopsa, solute_inter_hb, T, params):
    p = params
    f32 = jnp.float32

    x = jnp.concatenate([solvent_x, solute_x], axis=0)
    x = jnp.pad(x, ((0, N2P - N2), (0, 0)))
    batchf = jnp.concatenate([solvent_batch.astype(f32),
                              solute_batch.astype(f32) + 128.0])
    batchf2d = jnp.pad(batchf, (0, N2P - N2),
                       constant_values=float(NG)).reshape(N2P, 1)
    ea = jnp.concatenate([solvent_edge_attr, solute_edge_attr], axis=0)
    ea = jnp.pad(ea, ((0, E2P - E2), (0, 0)))
    row = jnp.concatenate([solvent_edge_index[0],
                           solute_edge_index[0] + N1]).astype(jnp.int32)
    col = jnp.concatenate([solvent_edge_index[1],
                           solute_edge_index[1] + N1]).astype(jnp.int32)
    row = jnp.pad(row, (0, E2P - E2), constant_values=N2)
    col = jnp.pad(col, (0, E2P - E2), constant_values=N2)
    u = jnp.concatenate([
        jnp.stack([solvent_ap, solvent_bp, solvent_topopsa], axis=1),
        jnp.stack([solute_ap, solute_bp, solute_topopsa], axis=1)], axis=0)
    u8 = jnp.pad(u, ((0, 0), (0, 5)))

    w1 = p["edge1"]["l1"]["w"]
    w1a, w1b, w1c = w1[0:128], w1[128:256], w1[256:272]
    w1u8 = _pad8(w1[272:275])
    b1r = p["edge1"]["l1"]["b"][None, :]
    w21 = p["edge1"]["l2"]["w"]
    b21 = p["edge1"]["l2"]["b"][None, :]

    wn = p["node1"]["l1"]["w"]
    n1w = (wn[0:128], wn[128:192], _pad8(wn[192:195]),
           p["node1"]["l1"]["b"][None, :], p["node1"]["l2"]["w"],
           p["node1"]["l2"]["b"][None, :])
    wg = p["glob1"]["l1"]["w"]
    g1w = (_pad8(wg[0:3]), wg[3:67], wg[67:131], p["glob1"]["l1"]["b"][None, :],
           p["glob1"]["l2"]["w"], p["glob1"]["l2"]["b"][None, :])
    gn1 = p["gnorm1"]
    w2 = p["edge2"]["l1"]["w"]
    w2a, w2b, w2c, w2u = w2[0:64], w2[64:128], w2[128:192], w2[192:256]
    b2e = p["edge2"]["l1"]["b"][None, :]
    w22 = p["edge2"]["l2"]["w"]
    b22 = p["edge2"]["l2"]["b"][None, :]
    wn2 = p["node2"]["l1"]["w"]
    n2w = (wn2[0:64], wn2[64:128], wn2[128:192],
           p["node2"]["l1"]["b"][None, :], p["node2"]["l2"]["w"],
           p["node2"]["l2"]["b"][None, :])
    wg2 = p["glob2"]["l1"]["w"]
    g2w = (wg2[0:64], wg2[64:128], wg2[128:192], p["glob2"]["l1"]["b"][None, :],
           p["glob2"]["l2"]["w"], p["glob2"]["l2"]["b"][None, :])
    gn2 = p["gnorm2"]

    a1p, b1t = _tc_pre(x, batchf2d, u8, w1a, w1b, w1u8, b1r)
    c1 = _tc_c1(ea, w1c)
    s1, r1 = _sc_pass1(a1p, b1t, c1, row, col)
    cnt = _sc_cnt(col)

    p8 = (w21, b21, *n1w, *g1w,
          gn1["mean_scale"][None, :], gn1["weight"][None, :],
          w2u, b2e, w2c)
    x1, msc, rw, u1, u2t, gstats = _tc_stage1(x, batchf2d, s1, cnt, u8, p8)

    xn1, a2p, b2t = _tc_stage1b(x1, batchf2d, msc, rw,
                                gn1["bias"][None, :], w2a, w2b, u2t)
    r1m = _tc_r1m(r1, w21, w2c)
    s2 = _sc_pass2(a2p, b2t, r1m, row, col)

    p8b = (w22, b22, *n2w, *g2w,
           gn2["mean_scale"][None, :], gn2["weight"][None, :])
    x2, msc2, rw2, u2 = _tc_stage2(xn1, batchf2d, s2, cnt, u1, gstats, p8b)

    nf = _tc_stage2b(x2, batchf2d, msc2, rw2, gn2["bias"][None, :], u2, gstats)

    return _tc_system(
        nf, solvent_inter_hb[:, None], solute_inter_hb[:, None],
        p["mpnn_proj"]["w"], p["mpnn_proj"]["b"][None, :],
        p["mpnn_e1"]["w"], p["mpnn_e1"]["b"][None, :],
        p["mpnn_e2"]["w"].reshape(32 * 128, 128),
        p["mpnn_e2"]["b"].reshape(128, 128),
        p["mpnn_root"], p["mpnn_bias"][None, :],
        p["gru_w_ih"], p["gru_w_hh"],
        p["gru_b_ih"][None, :], p["gru_b_hh"][None, :])


# speedup vs baseline: 6.0226x; 1.2334x over previous
import functools

import jax
import jax.numpy as jnp
from jax import lax
from jax.experimental import pallas as pl
from jax.experimental.pallas import tpu as pltpu
from jax.experimental.pallas import tpu_sc as plsc

V_IN = 128
E_IN = 16
H = 64
NG = 256
N1 = 10000
N2 = 20000
N2P = 20480
E1 = 320000
E2 = 640000
K = 128
CH = 157
EPT = K * CH
NTILES = 32
E2P = EPT * NTILES
NB = N2P // 1024
EB = 4096
NBE = E2P // EB
STRIPE = N2P // 16


def _bspec(shape, imap):
    return pl.BlockSpec(shape, imap)


def _full(shape):
    return pl.BlockSpec(shape, lambda i: tuple(0 for _ in shape))


def _pre_body(x_ref, b_ref, u8_ref, w1a_ref, w1b_ref, w1u_ref, b1_ref,
              *ab_ref):
    bb = b_ref[...]
    oneh = (bb == lax.broadcasted_iota(jnp.int32, (1024, NG), 1).astype(jnp.float32)).astype(jnp.float32)
    u1 = _dot(u8_ref[...], w1u_ref[...]) + b1_ref[...]
    x = x_ref[...]
    ab_ref[0][...] = _dot(x, w1a_ref[...]) + \
        _dot(oneh, u1)
    ab_ref[1][...] = _dot(x, w1b_ref[...])


def _tc_pre(x, batchf2d, u8, w1a, w1b, w1u8, b1r):
    return pl.pallas_call(
        _pre_body,
        grid=(NB,),
        in_specs=[
            _bspec((1024, V_IN), lambda i: (i, 0)),
            _bspec((1024, 1), lambda i: (i, 0)),
            _full((NG, 8)), _full((V_IN, H)), _full((V_IN, H)),
            _full((8, H)), _full((1, H)),
        ],
        out_shape=[jax.ShapeDtypeStruct((N2P, H), jnp.float32),
                   jax.ShapeDtypeStruct((N2P, H), jnp.float32)],
        out_specs=[_bspec((1024, H), lambda i: (i, 0)),
                   _bspec((1024, H), lambda i: (i, 0))],
    )(x, batchf2d, u8, w1a, w1b, w1u8, b1r)


def _c1_body(ea_ref, w_ref, o_ref):
    o_ref[...] = _dot(ea_ref[...], w_ref[...])


def _tc_c1(ea, w1c):
    return pl.pallas_call(
        _c1_body,
        grid=(NBE,),
        in_specs=[_bspec((EB, E_IN), lambda i: (i, 0)), _full((E_IN, H))],
        out_shape=jax.ShapeDtypeStruct((E2P, H), jnp.float32),
        out_specs=_bspec((EB, H), lambda i: (i, 0)),
    )(ea, w1c)


def _r1m_body(r_ref, w21_ref, w2c_ref, o_ref):
    m2 = _dot(w21_ref[...], w2c_ref[...])
    o_ref[...] = _dot(r_ref[...], m2)


def _tc_r1m(r1, w21, w2c):
    return pl.pallas_call(
        _r1m_body,
        grid=(NBE,),
        in_specs=[_bspec((EB, H), lambda i: (i, 0)), _full((H, H)), _full((H, H))],
        out_shape=jax.ShapeDtypeStruct((E2P, H), jnp.float32),
        out_specs=_bspec((EB, H), lambda i: (i, 0)),
    )(r1, w21, w2c)


def _dot(a, b):
    return jnp.dot(a, b, preferred_element_type=jnp.float32,
                   precision=lax.Precision.HIGHEST)


def _dotT(a, b):
    return lax.dot_general(a, b, (((0,), (0,)), ((), ())),
                           preferred_element_type=jnp.float32,
                           precision=lax.Precision.HIGHEST)


def _stage1_body(x_ref, b_ref, s_ref, cnt_ref, u8_ref,
                 w21_ref, b21_ref,
                 wnx_ref, wna_ref, wnu_ref, bn_ref, wn2_ref, bn2_ref,
                 wgu_ref, wgn_ref, wge_ref, bg_ref, wg2_ref, bg2_ref,
                 ms_ref, gw_ref,
                 w2u_ref, b2e_ref, w2c_ref,
                 x1_ref, msc_ref, rw_ref, u1_ref, u2t_ref, gst_ref,
                 acc_sum, acc_sq, acc_eg, acc_g):
    i = pl.program_id(0)

    @pl.when(i == 0)
    def _():
        acc_sum[...] = jnp.zeros_like(acc_sum)
        acc_sq[...] = jnp.zeros_like(acc_sq)
        acc_eg[...] = jnp.zeros_like(acc_eg)
        acc_g[...] = jnp.zeros_like(acc_g)

    bb = b_ref[...]
    oneh = (bb == lax.broadcasted_iota(jnp.int32, (1024, NG), 1).astype(jnp.float32)).astype(jnp.float32)
    s = s_ref[0] + s_ref[1]
    cnt = (cnt_ref[0, :, 0:1] + cnt_ref[1, :, 0:1])
    agg = _dot(s, w21_ref[...]) + cnt * b21_ref[...]
    un = _dot(u8_ref[...], wnu_ref[...]) + bn_ref[...]
    nh = jnp.maximum(_dot(x_ref[...], wnx_ref[...]) + _dot(agg, wna_ref[...]) +
                     _dot(oneh, un), 0.0)
    x1 = _dot(nh, wn2_ref[...]) + bn2_ref[...]
    x1_ref[...] = x1

    acc_sum[...] += _dotT(oneh, x1)
    acc_sq[...] += _dotT(oneh, x1 * x1)
    acc_eg[...] += _dotT(oneh, s)
    ones_col = jnp.ones((1024, 1), jnp.float32)
    acc_g[:, 0:1] += _dotT(oneh, cnt)
    acc_g[:, 1:2] += _dotT(oneh, ones_col)

    @pl.when(i == NB - 1)
    def _():
        ecnt = acc_g[:, 0:1]
        ncnt = jnp.maximum(acc_g[:, 1:2], 1.0)
        m = acc_sum[...] / ncnt
        q = acc_sq[...] / ncnt
        ms = ms_ref[...]
        var = q - (2.0 * ms - ms * ms) * m * m
        rstd = lax.rsqrt(var + 1e-5)
        edge_agg = (_dot(acc_eg[...], w21_ref[...]) + ecnt * b21_ref[...]) / \
            jnp.maximum(ecnt, 1.0)
        gh = jnp.maximum(_dot(u8_ref[...], wgu_ref[...]) + _dot(m, wgn_ref[...]) +
                         _dot(edge_agg, wge_ref[...]) + bg_ref[...], 0.0)
        u1 = _dot(gh, wg2_ref[...]) + bg2_ref[...]
        u1_ref[...] = u1
        msc_ref[...] = m * ms
        rw_ref[...] = rstd * gw_ref[...]
        u2t_ref[...] = _dot(u1, w2u_ref[...]) + b2e_ref[...] + \
            _dot(b21_ref[...], w2c_ref[...])
        gst_ref[...] = acc_g[...]


def _tc_stage1(x, batchf2d, s2p, cnt2p, u8, p8):
    outs = pl.pallas_call(
        _stage1_body,
        grid=(NB,),
        in_specs=[
            _bspec((1024, V_IN), lambda i: (i, 0)),
            _bspec((1024, 1), lambda i: (i, 0)),
            _bspec((2, 1024, H), lambda i: (0, i, 0)),
            _bspec((2, 1024, 16), lambda i: (0, i, 0)),
            _full((NG, 8)),
            _full((H, H)), _full((1, H)),
            _full((V_IN, H)), _full((H, H)), _full((8, H)), _full((1, H)),
            _full((H, H)), _full((1, H)),
            _full((8, H)), _full((H, H)), _full((H, H)), _full((1, H)),
            _full((H, H)), _full((1, H)),
            _full((1, H)), _full((1, H)),
            _full((H, H)), _full((1, H)), _full((H, H)),
        ],
        out_shape=[jax.ShapeDtypeStruct((N2P, H), jnp.float32),
                   jax.ShapeDtypeStruct((NG, H), jnp.float32),
                   jax.ShapeDtypeStruct((NG, H), jnp.float32),
                   jax.ShapeDtypeStruct((NG, H), jnp.float32),
                   jax.ShapeDtypeStruct((NG, H), jnp.float32),
                   jax.ShapeDtypeStruct((NG, 128), jnp.float32)],
        out_specs=[_bspec((1024, H), lambda i: (i, 0)),
                   _full((NG, H)), _full((NG, H)), _full((NG, H)),
                   _full((NG, H)), _full((NG, 128))],
        scratch_shapes=[pltpu.VMEM((NG, H), jnp.float32),
                        pltpu.VMEM((NG, H), jnp.float32),
                        pltpu.VMEM((NG, H), jnp.float32),
                        pltpu.VMEM((NG, 128), jnp.float32)],
    )(x, batchf2d, s2p, cnt2p, u8, *p8)
    return outs


def _stage1b_body(x1_ref, b_ref, msc_ref, rw_ref, bgn_ref, w2a_ref, w2b_ref,
                  u2t_ref, xn1_ref, *ab2_ref):
    bb = b_ref[...]
    oneh = (bb == lax.broadcasted_iota(jnp.int32, (1024, NG), 1).astype(jnp.float32)).astype(jnp.float32)
    xn1 = (x1_ref[...] - _dot(oneh, msc_ref[...])) * _dot(oneh, rw_ref[...]) + \
        bgn_ref[...]
    xn1_ref[...] = xn1
    ab2_ref[0][...] = _dot(xn1, w2a_ref[...]) + _dot(oneh, u2t_ref[...])
    ab2_ref[1][...] = _dot(xn1, w2b_ref[...])


def _tc_stage1b(x1, batchf2d, msc, rw, bgn, w2a, w2b, u2t):
    return pl.pallas_call(
        _stage1b_body,
        grid=(NB,),
        in_specs=[
            _bspec((1024, H), lambda i: (i, 0)),
            _bspec((1024, 1), lambda i: (i, 0)),
            _full((NG, H)), _full((NG, H)), _full((1, H)),
            _full((H, H)), _full((H, H)), _full((NG, H)),
        ],
        out_shape=[jax.ShapeDtypeStruct((N2P, H), jnp.float32),
                   jax.ShapeDtypeStruct((N2P, H), jnp.float32),
                   jax.ShapeDtypeStruct((N2P, H), jnp.float32)],
        out_specs=[_bspec((1024, H), lambda i: (i, 0)),
                   _bspec((1024, H), lambda i: (i, 0)),
                   _bspec((1024, H), lambda i: (i, 0))],
    )(x1, batchf2d, msc, rw, bgn, w2a, w2b, u2t)


def _stage2_body(xn1_ref, b_ref, s_ref, cnt_ref, u1_ref, gst_ref,
                 w22_ref, b22_ref,
                 wnx_ref, wna_ref, wnu_ref, bn_ref, wn2_ref, bn2_ref,
                 wgu_ref, wgn_ref, wge_ref, bg_ref, wg2_ref, bg2_ref,
                 ms_ref, gw_ref,
                 x2_ref, msc_ref, rw_ref, u2_ref,
                 acc_sum, acc_sq, acc_eg):
    i = pl.program_id(0)

    @pl.when(i == 0)
    def _():
        acc_sum[...] = jnp.zeros_like(acc_sum)
        acc_sq[...] = jnp.zeros_like(acc_sq)
        acc_eg[...] = jnp.zeros_like(acc_eg)

    bb = b_ref[...]
    oneh = (bb == lax.broadcasted_iota(jnp.int32, (1024, NG), 1).astype(jnp.float32)).astype(jnp.float32)
    s = s_ref[0] + s_ref[1]
    cnt = (cnt_ref[0, :, 0:1] + cnt_ref[1, :, 0:1])
    agg = _dot(s, w22_ref[...]) + cnt * b22_ref[...]
    un = _dot(u1_ref[...], wnu_ref[...]) + bn_ref[...]
    nh = jnp.maximum(_dot(xn1_ref[...], wnx_ref[...]) + _dot(agg, wna_ref[...]) +
                     _dot(oneh, un), 0.0)
    x2 = _dot(nh, wn2_ref[...]) + bn2_ref[...]
    x2_ref[...] = x2

    acc_sum[...] += _dotT(oneh, x2)
    acc_sq[...] += _dotT(oneh, x2 * x2)
    acc_eg[...] += _dotT(oneh, s)

    @pl.when(i == NB - 1)
    def _():
        ecnt = gst_ref[:, 0:1]
        ncnt = jnp.maximum(gst_ref[:, 1:2], 1.0)
        m = acc_sum[...] / ncnt
        q = acc_sq[...] / ncnt
        ms = ms_ref[...]
        var = q - (2.0 * ms - ms * ms) * m * m
        rstd = lax.rsqrt(var + 1e-5)
        edge_agg = (_dot(acc_eg[...], w22_ref[...]) + ecnt * b22_ref[...]) / \
            jnp.maximum(ecnt, 1.0)
        gh = jnp.maximum(_dot(u1_ref[...], wgu_ref[...]) + _dot(m, wgn_ref[...]) +
                         _dot(edge_agg, wge_ref[...]) + bg_ref[...], 0.0)
        u2 = _dot(gh, wg2_ref[...]) + bg2_ref[...]
        u2_ref[...] = u2
        msc_ref[...] = m * ms
        rw_ref[...] = rstd * gw_ref[...]


def _tc_stage2(xn1, batchf2d, s2p, cnt2p, u1, gstats, p8):
    return pl.pallas_call(
        _stage2_body,
        grid=(NB,),
        in_specs=[
            _bspec((1024, H), lambda i: (i, 0)),
            _bspec((1024, 1), lambda i: (i, 0)),
            _bspec((2, 1024, H), lambda i: (0, i, 0)),
            _bspec((2, 1024, 16), lambda i: (0, i, 0)),
            _full((NG, H)), _full((NG, 128)),
            _full((H, H)), _full((1, H)),
            _full((H, H)), _full((H, H)), _full((H, H)), _full((1, H)),
            _full((H, H)), _full((1, H)),
            _full((H, H)), _full((H, H)), _full((H, H)), _full((1, H)),
            _full((H, H)), _full((1, H)),
            _full((1, H)), _full((1, H)),
        ],
        out_shape=[jax.ShapeDtypeStruct((N2P, H), jnp.float32),
                   jax.ShapeDtypeStruct((NG, H), jnp.float32),
                   jax.ShapeDtypeStruct((NG, H), jnp.float32),
                   jax.ShapeDtypeStruct((NG, H), jnp.float32)],
        out_specs=[_bspec((1024, H), lambda i: (i, 0)),
                   _full((NG, H)), _full((NG, H)), _full((NG, H))],
        scratch_shapes=[pltpu.VMEM((NG, H), jnp.float32),
                        pltpu.VMEM((NG, H), jnp.float32),
                        pltpu.VMEM((NG, H), jnp.float32)],
    )(xn1, batchf2d, s2p, cnt2p, u1, gstats, *p8)


def _stage2b_body(x2_ref, b_ref, msc_ref, rw_ref, bgn_ref, u2_ref, gst_ref,
                  nf_ref, acc_xg):
    i = pl.program_id(0)

    @pl.when(i == 0)
    def _():
        acc_xg[...] = jnp.zeros_like(acc_xg)

    bb = b_ref[...]
    oneh = (bb == lax.broadcasted_iota(jnp.int32, (1024, NG), 1).astype(jnp.float32)).astype(jnp.float32)
    xn2 = (x2_ref[...] - _dot(oneh, msc_ref[...])) * _dot(oneh, rw_ref[...]) + \
        bgn_ref[...]
    acc_xg[...] += _dotT(oneh, xn2)

    @pl.when(i == NB - 1)
    def _():
        ncnt = jnp.maximum(gst_ref[:, 1:2], 1.0)
        xg = acc_xg[...] / ncnt
        nf_ref[:, 0:H] = xg
        nf_ref[:, H:2 * H] = u2_ref[...]


def _tc_stage2b(x2, batchf2d, msc2, rw2, bgn2, u2, gstats):
    return pl.pallas_call(
        _stage2b_body,
        grid=(NB,),
        in_specs=[
            _bspec((1024, H), lambda i: (i, 0)),
            _bspec((1024, 1), lambda i: (i, 0)),
            _full((NG, H)), _full((NG, H)), _full((1, H)),
            _full((NG, H)), _full((NG, 128)),
        ],
        out_shape=jax.ShapeDtypeStruct((NG, 128), jnp.float32),
        out_specs=_full((NG, 128)),
        scratch_shapes=[pltpu.VMEM((NG, H), jnp.float32)],
    )(x2, batchf2d, msc2, rw2, bgn2, u2, gstats)


def _system_body(nf_ref, sih_ref, tih_ref,
                 pw_ref, pb_ref, e1w_ref, e1b_ref, e2w_ref, e2br_ref,
                 root_ref, mb_ref, wih_ref, whh_ref, bih_ref, bhh_ref,
                 out_ref):
    D = 128
    h = jnp.maximum(_dot(nf_ref[...], pw_ref[...]) + pb_ref[...], 0.0)
    efa = jnp.concatenate([sih_ref[...], sih_ref[...]], axis=0)
    efb = jnp.concatenate([sih_ref[...], tih_ref[...]], axis=0)
    eha = jnp.maximum(efa * e1w_ref[...] + e1b_ref[...], 0.0)
    ehb = jnp.maximum(efb * e1w_ref[...] + e1b_ref[...], 0.0)
    hb = _dot(h, e2br_ref[...])
    msga = hb
    msgb = hb
    for k in range(32):
        qk = _dot(h, e2w_ref[pl.ds(k * D, D), :])
        msga = msga + eha[:, k:k + 1] * qk
        msgb = msgb + ehb[:, k:k + 1] * qk
    aggm = jnp.concatenate([msga[128:256], msga[0:128]], axis=0) + msgb
    out = jnp.maximum(_dot(h, root_ref[...]) + aggm + mb_ref[...], 0.0)
    gi = lax.dot_general(out, wih_ref[...], (((1,), (1,)), ((), ())),
                         preferred_element_type=jnp.float32,
                         precision=lax.Precision.HIGHEST) + bih_ref[...]
    gh = lax.dot_general(h, whh_ref[...], (((1,), (1,)), ((), ())),
                         preferred_element_type=jnp.float32,
                         precision=lax.Precision.HIGHEST) + bhh_ref[...]
    r = jax.nn.sigmoid(gi[:, 0:D] + gh[:, 0:D])
    z = jax.nn.sigmoid(gi[:, D:2 * D] + gh[:, D:2 * D])
    nh = jnp.tanh(gi[:, 2 * D:3 * D] + r * gh[:, 2 * D:3 * D])
    out_ref[...] = (1.0 - z) * nh + z * h


def _tc_system(nf, sih2, tih2, pw, pb, e1w, e1b, e2w, e2br, root, mb,
               wih, whh, bih, bhh):
    return pl.pallas_call(
        _system_body,
        grid=(1,),
        in_specs=[
            _full((NG, 128)), _full((128, 1)), _full((128, 1)),
            _full((128, 128)), _full((1, 128)), _full((1, 32)), _full((1, 32)),
            _full((4096, 128)), _full((128, 128)),
            _full((128, 128)), _full((1, 128)),
            _full((384, 128)), _full((384, 128)), _full((1, 384)), _full((1, 384)),
        ],
        out_shape=jax.ShapeDtypeStruct((NG, 128), jnp.float32),
        out_specs=_full((NG, 128)),
    )(nf, sih2, tih2, pw, pb, e1w, e1b, e2w, e2br, root, mb, wih, whh, bih, bhh)


@functools.lru_cache(maxsize=None)
def _make_sc_fused(store_r: bool):
    mesh = plsc.VectorSubcoreMesh(core_axis_name="c", subcore_axis_name="s")
    outs = [jax.ShapeDtypeStruct((2, N2P, H), jnp.float32)]
    if store_r:
        outs.append(jax.ShapeDtypeStruct((E2P, H), jnp.float32))

    @functools.partial(
        pl.kernel, mesh=mesh, out_type=outs,
        compiler_params=pltpu.CompilerParams(use_tc_tiling_on_sc=False),
        scratch_types=[
            pltpu.VMEM((K,), jnp.int32),
            pltpu.VMEM((K,), jnp.int32),
            pltpu.VMEM((K, H), jnp.float32),
            pltpu.VMEM((K, H), jnp.float32),
            pltpu.VMEM((K, H), jnp.float32),
            pltpu.VMEM_SHARED((N2P, H), jnp.float32),
            pltpu.SemaphoreType.DMA,
            pltpu.SemaphoreType.DMA,
            pltpu.SemaphoreType.DMA,
        ])
    def sc_fused(a_hbm, b_hbm, c_hbm, row_hbm, col_hbm, *rest):
        it = iter(rest)
        s_out = next(it)
        r_out = next(it) if store_r else None
        idx_r = next(it); idx_c = next(it)
        abuf = next(it); bbuf = next(it); obuf = next(it)
        s_sh = next(it)
        sem = next(it); sem2 = next(it); sem3 = next(it)

        c = lax.axis_index("c")
        sid = lax.axis_index("s")
        wid = sid * 2 + c

        def zrow(i, _):
            for j in range(H // 16):
                obuf[i, pl.ds(j * 16, 16)] = jnp.zeros((16,), jnp.float32)
            return 0
        lax.fori_loop(0, K, zrow, 0)
        r0 = sid * STRIPE
        for t in range(STRIPE // K):
            pltpu.sync_copy(obuf, s_sh.at[pl.ds(r0 + t * K, K)])
        plsc.subcore_barrier()

        def chunk(ch, _):
            base = pl.multiple_of(wid * EPT + ch * K, K)
            i1 = pltpu.async_copy(row_hbm.at[pl.ds(base, K)], idx_r, sem)
            i2 = pltpu.async_copy(col_hbm.at[pl.ds(base, K)], idx_c, sem2)
            g3 = pltpu.async_copy(c_hbm.at[pl.ds(base, K)], obuf, sem3)
            i1.wait(); i2.wait()
            g1 = pltpu.async_copy(a_hbm.at[idx_r], abuf, sem)
            g2 = pltpu.async_copy(b_hbm.at[idx_c], bbuf, sem2)
            g1.wait(); g2.wait(); g3.wait()

            def erow(i, _):
                for j in range(H // 16):
                    sl = pl.ds(j * 16, 16)
                    obuf[i, sl] = jnp.maximum(
                        abuf[i, sl] + bbuf[i, sl] + obuf[i, sl], 0.0)
                return 0
            lax.fori_loop(0, K, erow, 0)
            if store_r:
                s1 = pltpu.async_copy(obuf, r_out.at[pl.ds(base, K)], sem)
            s2 = pltpu.async_copy(obuf, s_sh.at[idx_c], sem2, add=True)
            if store_r:
                s1.wait()
            s2.wait()
            return 0
        lax.fori_loop(0, CH, chunk, 0)

        plsc.subcore_barrier()
        pltpu.sync_copy(s_sh.at[pl.ds(r0, STRIPE)], s_out.at[c, pl.ds(r0, STRIPE)])

    return sc_fused


@functools.lru_cache(maxsize=None)
def _make_sc_cnt():
    mesh = plsc.VectorSubcoreMesh(core_axis_name="c", subcore_axis_name="s")

    @functools.partial(
        pl.kernel, mesh=mesh,
        out_type=jax.ShapeDtypeStruct((2, N2P, 16), jnp.float32),
        compiler_params=pltpu.CompilerParams(use_tc_tiling_on_sc=False),
        scratch_types=[
            pltpu.VMEM((K,), jnp.int32),
            pltpu.VMEM((K, 16), jnp.float32),
            pltpu.VMEM((K, 16), jnp.float32),
            pltpu.VMEM_SHARED((N2P, 16), jnp.float32),
        ])
    def sc_cnt(col_hbm, cnt_out, idx_c, ones, zc, cnt_sh):
        c = lax.axis_index("c")
        sid = lax.axis_index("s")
        wid = sid * 2 + c

        def zrow(i, _):
            zc[i, pl.ds(0, 16)] = jnp.zeros((16,), jnp.float32)
            ones[i, pl.ds(0, 16)] = jnp.full((16,), 1.0, jnp.float32)
            return 0
        lax.fori_loop(0, K, zrow, 0)
        r0 = sid * STRIPE
        for t in range(STRIPE // K):
            pltpu.sync_copy(zc, cnt_sh.at[pl.ds(r0 + t * K, K)])
        plsc.subcore_barrier()

        def chunk(ch, _):
            base = pl.multiple_of(wid * EPT + ch * K, K)
            pltpu.sync_copy(col_hbm.at[pl.ds(base, K)], idx_c)
            pltpu.sync_copy(ones, cnt_sh.at[idx_c], add=True)
            return 0
        lax.fori_loop(0, CH, chunk, 0)

        plsc.subcore_barrier()
        pltpu.sync_copy(cnt_sh.at[pl.ds(r0, STRIPE)],
                        cnt_out.at[c, pl.ds(r0, STRIPE)])

    return sc_cnt


def _sc_pass1(a, b, cc, row, col):
    s, r = _make_sc_fused(True)(a, b, cc, row, col)
    return s, r


def _sc_pass2(a, b, cc, row, col):
    s, _ = _make_sc_fused(True)(a, b, cc, row, col)
    return s


def _sc_cnt(col):
    return _make_sc_cnt()(col)


def _pad8(w):
    return jnp.pad(w, ((0, 8 - w.shape[0]), (0, 0)))


def kernel(solvent_x, solvent_edge_index, solvent_edge_attr, solvent_batch,
           solvent_ap, solvent_bp, solvent_topopsa, solvent_inter_hb,
           solute_x, solute_edge_index, solute_edge_attr, solute_batch,
           solute_ap, solute_bp, solute_topopsa, solute_inter_hb, T, params):
    p = params
    f32 = jnp.float32

    x = jnp.concatenate([solvent_x, solute_x], axis=0)
    x = jnp.pad(x, ((0, N2P - N2), (0, 0)))
    batchf = jnp.concatenate([solvent_batch.astype(f32),
                              solute_batch.astype(f32) + 128.0])
    batchf2d = jnp.pad(batchf, (0, N2P - N2),
                       constant_values=float(NG)).reshape(N2P, 1)
    ea = jnp.concatenate([solvent_edge_attr, solute_edge_attr], axis=0)
    ea = jnp.pad(ea, ((0, E2P - E2), (0, 0)))
    row = jnp.concatenate([solvent_edge_index[0],
                           solute_edge_index[0] + N1]).astype(jnp.int32)
    col = jnp.concatenate([solvent_edge_index[1],
                           solute_edge_index[1] + N1]).astype(jnp.int32)
    row = jnp.pad(row, (0, E2P - E2), constant_values=N2)
    col = jnp.pad(col, (0, E2P - E2), constant_values=N2)
    u = jnp.concatenate([
        jnp.stack([solvent_ap, solvent_bp, solvent_topopsa], axis=1),
        jnp.stack([solute_ap, solute_bp, solute_topopsa], axis=1)], axis=0)
    u8 = jnp.pad(u, ((0, 0), (0, 5)))

    w1 = p["edge1"]["l1"]["w"]
    w1a, w1b, w1c = w1[0:128], w1[128:256], w1[256:272]
    w1u8 = _pad8(w1[272:275])
    b1r = p["edge1"]["l1"]["b"][None, :]
    w21 = p["edge1"]["l2"]["w"]
    b21 = p["edge1"]["l2"]["b"][None, :]

    wn = p["node1"]["l1"]["w"]
    n1w = (wn[0:128], wn[128:192], _pad8(wn[192:195]),
           p["node1"]["l1"]["b"][None, :], p["node1"]["l2"]["w"],
           p["node1"]["l2"]["b"][None, :])
    wg = p["glob1"]["l1"]["w"]
    g1w = (_pad8(wg[0:3]), wg[3:67], wg[67:131], p["glob1"]["l1"]["b"][None, :],
           p["glob1"]["l2"]["w"], p["glob1"]["l2"]["b"][None, :])
    gn1 = p["gnorm1"]
    w2 = p["edge2"]["l1"]["w"]
    w2a, w2b, w2c, w2u = w2[0:64], w2[64:128], w2[128:192], w2[192:256]
    b2e = p["edge2"]["l1"]["b"][None, :]
    w22 = p["edge2"]["l2"]["w"]
    b22 = p["edge2"]["l2"]["b"][None, :]
    wn2 = p["node2"]["l1"]["w"]
    n2w = (wn2[0:64], wn2[64:128], wn2[128:192],
           p["node2"]["l1"]["b"][None, :], p["node2"]["l2"]["w"],
           p["node2"]["l2"]["b"][None, :])
    wg2 = p["glob2"]["l1"]["w"]
    g2w = (wg2[0:64], wg2[64:128], wg2[128:192], p["glob2"]["l1"]["b"][None, :],
           p["glob2"]["l2"]["w"], p["glob2"]["l2"]["b"][None, :])
    gn2 = p["gnorm2"]

    a1p, b1t = _tc_pre(x, batchf2d, u8, w1a, w1b, w1u8, b1r)
    c1 = _tc_c1(ea, w1c)
    s1, r1 = _sc_pass1(a1p, b1t, c1, row, col)
    cnt = _sc_cnt(col)

    p8 = (w21, b21, *n1w, *g1w,
          gn1["mean_scale"][None, :], gn1["weight"][None, :],
          w2u, b2e, w2c)
    x1, msc, rw, u1, u2t, gstats = _tc_stage1(x, batchf2d, s1, cnt, u8, p8)

    xn1, a2p, b2t = _tc_stage1b(x1, batchf2d, msc, rw,
                                gn1["bias"][None, :], w2a, w2b, u2t)
    r1m = _tc_r1m(r1, w21, w2c)
    s2 = _sc_pass2(a2p, b2t, r1m, row, col)

    p8b = (w22, b22, *n2w, *g2w,
           gn2["mean_scale"][None, :], gn2["weight"][None, :])
    x2, msc2, rw2, u2 = _tc_stage2(xn1, batchf2d, s2, cnt, u1, gstats, p8b)

    nf = _tc_stage2b(x2, batchf2d, msc2, rw2, gn2["bias"][None, :], u2, gstats)

    return _tc_system(
        nf, solvent_inter_hb[:, None], solute_inter_hb[:, None],
        p["mpnn_proj"]["w"], p["mpnn_proj"]["b"][None, :],
        p["mpnn_e1"]["w"], p["mpnn_e1"]["b"][None, :],
        p["mpnn_e2"]["w"].reshape(32 * 128, 128),
        p["mpnn_e2"]["b"].reshape(128, 128),
        p["mpnn_root"], p["mpnn_bias"][None, :],
        p["gru_w_ih"], p["gru_w_hh"],
        p["gru_b_ih"][None, :], p["gru_b_hh"][None, :])
